# Initial kernel scaffold; baseline (speedup 1.0000x reference)
import jax
import jax.numpy as jnp
from jax.experimental import pallas as pl


def kernel(node_emb, edge_emb, edge_index, attn_Wqkv_w, attn_Wqkv_b, attn_out_w, attn_out_b, out_proj_w, out_proj_b, g_key_w, g_key_b, g_query_w, g_query_b, g_value_w, g_value_b, g_edge_w, g_skip_w, g_skip_b):
    raise NotImplementedError("write your pallas kernel here")


# speedup vs baseline: 6.5497x; 6.5497x over previous
import functools
import math

import jax
import jax.numpy as jnp
from jax.experimental import pallas as pl
from jax.experimental.pallas import tpu as pltpu

B, M, N, D, H = 4, 20, 480, 128, 2
MN = M + N
HD = D // H


def _mha_kernel(x_ref, wq0, wq1, wk0, wk1, wv0, wv1, wo0, wo1,
                bq0, bq1, bk0, bk1, bv0, bv1, bo, ee_ref):
    x = x_ref[0]
    scale = 1.0 / math.sqrt(HD)
    o_parts = []
    for wq, wk, wv, bq, bk, bv in ((wq0, wk0, wv0, bq0, bk0, bv0),
                                   (wq1, wk1, wv1, bq1, bk1, bv1)):
        q = jnp.dot(x, wq[...], preferred_element_type=jnp.float32) + bq[...]
        k = jnp.dot(x, wk[...], preferred_element_type=jnp.float32) + bk[...]
        v = jnp.dot(x, wv[...], preferred_element_type=jnp.float32) + bv[...]
        s = jax.lax.dot_general(q, k, (((1,), (1,)), ((), ())),
                                preferred_element_type=jnp.float32) * scale
        m_ = jnp.max(s, axis=1, keepdims=True)
        e = jnp.exp(s - m_)
        p = e / jnp.sum(e, axis=1, keepdims=True)
        o_parts.append(jnp.dot(p, v, preferred_element_type=jnp.float32))
    out = (jnp.dot(o_parts[0], wo0[...], preferred_element_type=jnp.float32)
           + jnp.dot(o_parts[1], wo1[...], preferred_element_type=jnp.float32)
           + bo[...])
    ee_ref[0] = out


def _graph_kernel(ee_ref, e2src_ref, agent_ref, cust_ref,
                  wq, bq, wk, bk, wv, bv, we, ws, bs, wo, bo,
                  out_ref, eeout_ref):
    ea = ee_ref[0]
    agent = agent_ref[0]
    cust = cust_ref[0]
    EA = ea[:, M:, :]
    scale = 1.0 / math.sqrt(D)

    e1 = jnp.dot(EA.reshape(M * N, D), we[...],
                 preferred_element_type=jnp.float32).reshape(M, N, D)
    e2 = jnp.dot(e2src_ref[0].reshape(M * N, D), we[...],
                 preferred_element_type=jnp.float32).reshape(M, N, D)

    q_a = jnp.dot(agent, wq[...], preferred_element_type=jnp.float32) + bq[...]
    k_a = jnp.dot(agent, wk[...], preferred_element_type=jnp.float32) + bk[...]
    v_a = jnp.dot(agent, wv[...], preferred_element_type=jnp.float32) + bv[...]
    q_c = jnp.dot(cust, wq[...], preferred_element_type=jnp.float32) + bq[...]
    k_c = jnp.dot(cust, wk[...], preferred_element_type=jnp.float32) + bk[...]
    v_c = jnp.dot(cust, wv[...], preferred_element_type=jnp.float32) + bv[...]

    alpha1 = (jax.lax.dot_general(k_a, q_c, (((1,), (1,)), ((), ())),
                                  preferred_element_type=jnp.float32)
              + jnp.sum(e1 * q_c[None, :, :], axis=-1)) * scale
    m1 = jnp.max(alpha1, axis=0, keepdims=True)
    ex1 = jnp.exp(alpha1 - m1)
    coef1 = ex1 / (jnp.sum(ex1, axis=0, keepdims=True) + 1e-16)
    agg1 = (jax.lax.dot_general(coef1, v_a, (((0,), (0,)), ((), ())),
                                preferred_element_type=jnp.float32)
            + jnp.sum(coef1[:, :, None] * e1, axis=0))
    cust_out = (agg1 + jnp.dot(cust, ws[...], preferred_element_type=jnp.float32)
                + bs[...] + cust)

    alpha2 = (jax.lax.dot_general(q_a, k_c, (((1,), (1,)), ((), ())),
                                  preferred_element_type=jnp.float32)
              + jnp.sum(e2 * q_a[:, None, :], axis=-1)) * scale
    m2 = jnp.max(alpha2, axis=1, keepdims=True)
    ex2 = jnp.exp(alpha2 - m2)
    coef2 = ex2 / (jnp.sum(ex2, axis=1, keepdims=True) + 1e-16)
    agg2 = (jnp.dot(coef2, v_c, preferred_element_type=jnp.float32)
            + jnp.sum(coef2[:, :, None] * e2, axis=1))
    agent_out = (agg2 + jnp.dot(agent, ws[...], preferred_element_type=jnp.float32)
                 + bs[...] + agent)

    out_ref[0, :M, :] = agent_out
    out_ref[0, M:, :] = cust_out

    ap = jnp.dot(agent_out, wo[...], preferred_element_type=jnp.float32)
    cp = jnp.dot(cust_out, wo[...], preferred_element_type=jnp.float32)
    eeout_ref[0, :, :M, :] = (jnp.broadcast_to(agent_out[None, :, :], (M, M, D))
                              + ea[:, :M, :])
    eeout_ref[0, :, M:, :] = (ap[:, None, :] + cp[None, :, :] + bo[...]
                              + ea[:, M:, :])


@jax.jit
def kernel(node_emb, edge_emb, edge_index,
           attn_Wqkv_w, attn_Wqkv_b, attn_out_w, attn_out_b,
           out_proj_w, out_proj_b,
           g_key_w, g_key_b, g_query_w, g_query_b,
           g_value_w, g_value_b, g_edge_w, g_skip_w, g_skip_b):
    f32 = jnp.float32
    x = edge_emb.reshape(B * M, MN, D)

    wqkv_t = attn_Wqkv_w.T
    wq0 = wqkv_t[:, 0:HD]
    wq1 = wqkv_t[:, HD:D]
    wk0 = wqkv_t[:, D:D + HD]
    wk1 = wqkv_t[:, D + HD:2 * D]
    wv0 = wqkv_t[:, 2 * D:2 * D + HD]
    wv1 = wqkv_t[:, 2 * D + HD:3 * D]
    bq0 = attn_Wqkv_b[0:HD].reshape(1, HD)
    bq1 = attn_Wqkv_b[HD:D].reshape(1, HD)
    bk0 = attn_Wqkv_b[D:D + HD].reshape(1, HD)
    bk1 = attn_Wqkv_b[D + HD:2 * D].reshape(1, HD)
    bv0 = attn_Wqkv_b[2 * D:2 * D + HD].reshape(1, HD)
    bv1 = attn_Wqkv_b[2 * D + HD:3 * D].reshape(1, HD)
    wo_t = attn_out_w.T
    wo0 = wo_t[:HD, :]
    wo1 = wo_t[HD:, :]
    bo_attn = attn_out_b.reshape(1, D)

    wspec = pl.BlockSpec(None)

    ee = pl.pallas_call(
        _mha_kernel,
        grid=(B * M,),
        in_specs=[pl.BlockSpec((1, MN, D), lambda i: (i, 0, 0))]
        + [wspec] * 15,
        out_specs=pl.BlockSpec((1, MN, D), lambda i: (i, 0, 0)),
        out_shape=jax.ShapeDtypeStruct((B * M, MN, D), f32),
        compiler_params=pltpu.CompilerParams(
            dimension_semantics=("parallel",)),
    )(x, wq0, wq1, wk0, wk1, wv0, wv1, wo0, wo1,
      bq0, bq1, bk0, bk1, bv0, bv1, bo_attn)

    ee4 = ee.reshape(B, M, MN, D)
    e2src = jnp.transpose(ee4[:, :, M:, :], (0, 2, 1, 3)).reshape(B, M, N, D)

    agent_in = node_emb[:, :M, :]
    cust_in = node_emb[:, M:, :]

    out, eeout = pl.pallas_call(
        _graph_kernel,
        grid=(B,),
        in_specs=[
            pl.BlockSpec((1, M, MN, D), lambda b: (b, 0, 0, 0)),
            pl.BlockSpec((1, M, N, D), lambda b: (b, 0, 0, 0)),
            pl.BlockSpec((1, M, D), lambda b: (b, 0, 0)),
            pl.BlockSpec((1, N, D), lambda b: (b, 0, 0)),
        ] + [wspec] * 11,
        out_specs=[
            pl.BlockSpec((1, MN, D), lambda b: (b, 0, 0)),
            pl.BlockSpec((1, M, MN, D), lambda b: (b, 0, 0, 0)),
        ],
        out_shape=[
            jax.ShapeDtypeStruct((B, MN, D), f32),
            jax.ShapeDtypeStruct((B, M, MN, D), f32),
        ],
        compiler_params=pltpu.CompilerParams(
            dimension_semantics=("parallel",)),
    )(ee4, e2src, agent_in, cust_in,
      g_query_w.T, g_query_b.reshape(1, D),
      g_key_w.T, g_key_b.reshape(1, D),
      g_value_w.T, g_value_b.reshape(1, D),
      g_edge_w.T, g_skip_w.T, g_skip_b.reshape(1, D),
      out_proj_w.T, out_proj_b.reshape(1, D))

    return out, eeout


# speedup vs baseline: 7.7320x; 1.1805x over previous
import math

import jax
import jax.numpy as jnp
from jax.experimental import pallas as pl
from jax.experimental.pallas import tpu as pltpu

B, M, N, D, H = 4, 20, 480, 128, 2
MN = M + N
HD = D // H


def _fused_kernel(x_ref, node_ref,
                  wq0, wq1, wk0, wk1, wv0, wv1, wo0, wo1,
                  bq0, bq1, bk0, bk1, bv0, bv1, boa,
                  gwq, gbq, gwk, gbk, gwv, gbv, gwe, gws, gbs, gwo, gbo,
                  out_ref, eeout_ref):
    mm = pl.program_id(1)

    @pl.when(mm < M)
    def _mha_phase():
        bf16 = jnp.bfloat16
        x = x_ref[0, 0].astype(bf16)
        scale = 1.0 / math.sqrt(HD)
        o_parts = []
        for wq, wk, wv, bq, bk, bv in ((wq0, wk0, wv0, bq0, bk0, bv0),
                                       (wq1, wk1, wv1, bq1, bk1, bv1)):
            q = jnp.dot(x, wq[...].astype(bf16),
                        preferred_element_type=jnp.float32) + bq[...]
            k = jnp.dot(x, wk[...].astype(bf16),
                        preferred_element_type=jnp.float32) + bk[...]
            v = jnp.dot(x, wv[...].astype(bf16),
                        preferred_element_type=jnp.float32) + bv[...]
            s = jax.lax.dot_general(q.astype(bf16), k.astype(bf16),
                                    (((1,), (1,)), ((), ())),
                                    preferred_element_type=jnp.float32) * scale
            m_ = jnp.max(s, axis=1, keepdims=True)
            e = jnp.exp(s - m_)
            p = e / jnp.sum(e, axis=1, keepdims=True)
            o_parts.append(jnp.dot(p.astype(bf16), v.astype(bf16),
                                   preferred_element_type=jnp.float32))
        out = (jnp.dot(o_parts[0].astype(bf16), wo0[...].astype(bf16),
                       preferred_element_type=jnp.float32)
               + jnp.dot(o_parts[1].astype(bf16), wo1[...].astype(bf16),
                         preferred_element_type=jnp.float32)
               + boa[...])
        eeout_ref[0, mm] = out

    @pl.when(mm == M)
    def _graph_phase():
        ea = eeout_ref[0]
        node = node_ref[0]
        agent = node[:M, :]
        cust = node[M:, :]
        EA = ea[:, M:, :]
        scale = 1.0 / math.sqrt(D)

        EB = jnp.transpose(EA, (1, 0, 2)).reshape(M, N, D)

        f32 = jnp.float32
        q_a = jnp.dot(agent, gwq[...], preferred_element_type=f32) + gbq[...]
        k_a = jnp.dot(agent, gwk[...], preferred_element_type=f32) + gbk[...]
        v_a = jnp.dot(agent, gwv[...], preferred_element_type=f32) + gbv[...]
        q_c = jnp.dot(cust, gwq[...], preferred_element_type=f32) + gbq[...]
        k_c = jnp.dot(cust, gwk[...], preferred_element_type=f32) + gbk[...]
        v_c = jnp.dot(cust, gwv[...], preferred_element_type=f32) + gbv[...]
        qe_c = jax.lax.dot_general(q_c, gwe[...], (((1,), (1,)), ((), ())),
                                   preferred_element_type=f32)
        qe_a = jax.lax.dot_general(q_a, gwe[...], (((1,), (1,)), ((), ())),
                                   preferred_element_type=f32)

        alpha1 = (jax.lax.dot_general(k_a, q_c, (((1,), (1,)), ((), ())),
                                      preferred_element_type=f32)
                  + jnp.sum(EA * qe_c[None, :, :], axis=-1)) * scale
        m1 = jnp.max(alpha1, axis=0, keepdims=True)
        ex1 = jnp.exp(alpha1 - m1)
        coef1 = ex1 / (jnp.sum(ex1, axis=0, keepdims=True) + 1e-16)
        wsum1 = jnp.sum(coef1[:, :, None] * EA, axis=0)
        agg1 = (jax.lax.dot_general(coef1, v_a, (((0,), (0,)), ((), ())),
                                    preferred_element_type=f32)
                + jnp.dot(wsum1, gwe[...], preferred_element_type=f32))
        cust_out = (agg1 + jnp.dot(cust, gws[...], preferred_element_type=f32)
                    + gbs[...] + cust)

        alpha2 = (jax.lax.dot_general(q_a, k_c, (((1,), (1,)), ((), ())),
                                      preferred_element_type=f32)
                  + jnp.sum(EB * qe_a[:, None, :], axis=-1)) * scale
        m2 = jnp.max(alpha2, axis=1, keepdims=True)
        ex2 = jnp.exp(alpha2 - m2)
        coef2 = ex2 / (jnp.sum(ex2, axis=1, keepdims=True) + 1e-16)
        wsum2 = jnp.sum(coef2[:, :, None] * EB, axis=1)
        agg2 = (jnp.dot(coef2, v_c, preferred_element_type=f32)
                + jnp.dot(wsum2, gwe[...], preferred_element_type=f32))
        agent_out = (agg2 + jnp.dot(agent, gws[...], preferred_element_type=f32)
                     + gbs[...] + agent)

        out_ref[0, :M, :] = agent_out
        out_ref[0, M:, :] = cust_out

        ap = jnp.dot(agent_out, gwo[...], preferred_element_type=f32)
        cp = jnp.dot(cust_out, gwo[...], preferred_element_type=f32)
        eeout_ref[0, :, :M, :] = (jnp.broadcast_to(agent_out[None, :, :],
                                                   (M, M, D)) + ea[:, :M, :])
        eeout_ref[0, :, M:, :] = (ap[:, None, :] + cp[None, :, :] + gbo[...]
                                  + ea[:, M:, :])


@jax.jit
def kernel(node_emb, edge_emb, edge_index,
           attn_Wqkv_w, attn_Wqkv_b, attn_out_w, attn_out_b,
           out_proj_w, out_proj_b,
           g_key_w, g_key_b, g_query_w, g_query_b,
           g_value_w, g_value_b, g_edge_w, g_skip_w, g_skip_b):
    f32 = jnp.float32

    wqkv_t = attn_Wqkv_w.T
    wq0 = wqkv_t[:, 0:HD]
    wq1 = wqkv_t[:, HD:D]
    wk0 = wqkv_t[:, D:D + HD]
    wk1 = wqkv_t[:, D + HD:2 * D]
    wv0 = wqkv_t[:, 2 * D:2 * D + HD]
    wv1 = wqkv_t[:, 2 * D + HD:3 * D]
    bq0 = attn_Wqkv_b[0:HD].reshape(1, HD)
    bq1 = attn_Wqkv_b[HD:D].reshape(1, HD)
    bk0 = attn_Wqkv_b[D:D + HD].reshape(1, HD)
    bk1 = attn_Wqkv_b[D + HD:2 * D].reshape(1, HD)
    bv0 = attn_Wqkv_b[2 * D:2 * D + HD].reshape(1, HD)
    bv1 = attn_Wqkv_b[2 * D + HD:3 * D].reshape(1, HD)
    wo_t = attn_out_w.T
    wo0 = wo_t[:HD, :]
    wo1 = wo_t[HD:, :]
    boa = attn_out_b.reshape(1, D)

    wspec = pl.BlockSpec(None)

    out, eeout = pl.pallas_call(
        _fused_kernel,
        grid=(B, M + 1),
        in_specs=[
            pl.BlockSpec((1, 1, MN, D), lambda b, mm: (b, mm % M, 0, 0)),
            pl.BlockSpec((1, MN, D), lambda b, mm: (b, 0, 0)),
        ] + [wspec] * 26,
        out_specs=[
            pl.BlockSpec((1, MN, D), lambda b, mm: (b, 0, 0)),
            pl.BlockSpec((1, M, MN, D), lambda b, mm: (b, 0, 0, 0)),
        ],
        out_shape=[
            jax.ShapeDtypeStruct((B, MN, D), f32),
            jax.ShapeDtypeStruct((B, M, MN, D), f32),
        ],
        compiler_params=pltpu.CompilerParams(
            dimension_semantics=("parallel", "arbitrary")),
    )(edge_emb, node_emb,
      wq0, wq1, wk0, wk1, wv0, wv1, wo0, wo1,
      bq0, bq1, bk0, bk1, bv0, bv1, boa,
      g_query_w.T, g_query_b.reshape(1, D),
      g_key_w.T, g_key_b.reshape(1, D),
      g_value_w.T, g_value_b.reshape(1, D),
      g_edge_w.T, g_skip_w.T, g_skip_b.reshape(1, D),
      out_proj_w.T, out_proj_b.reshape(1, D))

    return out, eeout
